# Initial kernel scaffold; baseline (speedup 1.0000x reference)
import jax
import jax.numpy as jnp
from jax.experimental import pallas as pl


def kernel(efeat, src_feat, dst_feat, src_idx, dst_idx, W_e, W_s, W_d, b, W_out, b_out, gamma, beta):
    raise NotImplementedError("write your pallas kernel here")


# speedup vs baseline: 3.1678x; 3.1678x over previous
import functools

import jax
import jax.numpy as jnp
from jax import lax
from jax.experimental import pallas as pl
from jax.experimental.pallas import tpu as pltpu
from jax.experimental.pallas import tpu_sc as plsc

NC = 2
NS = 16
NW = NC * NS
GRP = 128


def _tables_body(src_ref, dst_ref, wst_ref, wdt_ref, b_ref, ts_ref, td_ref):
    ts_ref[...] = jnp.dot(src_ref[...], wst_ref[...],
                          preferred_element_type=jnp.float32)
    td_ref[...] = jnp.dot(dst_ref[...], wdt_ref[...],
                          preferred_element_type=jnp.float32) + b_ref[...]


def _edge_body(ef_ref, g_ref, wet_ref, wot_ref, bo_ref, gam_ref, bet_ref,
               out_ref):
    s = jnp.dot(ef_ref[...], wet_ref[...],
                preferred_element_type=jnp.float32) + g_ref[...]
    h = s * jax.nn.sigmoid(s)
    o = jnp.dot(h, wot_ref[...], preferred_element_type=jnp.float32)
    o = o + bo_ref[...]
    mu = jnp.mean(o, axis=-1, keepdims=True)
    var = jnp.mean((o - mu) ** 2, axis=-1, keepdims=True)
    out_ref[...] = ((o - mu) * lax.rsqrt(var + 1e-5)) * gam_ref[...] + bet_ref[...]


def _sc_gather_sum(E, H):
    R = E // GRP
    steps = (R + NW - 1) // NW
    mesh = plsc.VectorSubcoreMesh(core_axis_name="c", subcore_axis_name="s")

    @functools.partial(
        pl.kernel,
        mesh=mesh,
        out_type=jax.ShapeDtypeStruct((R, GRP, H), jnp.float32),
        scratch_types=[
            pltpu.VMEM((GRP,), jnp.int32),
            pltpu.VMEM((GRP,), jnp.int32),
            pltpu.VMEM((GRP, H), jnp.float32),
            pltpu.VMEM((GRP, H), jnp.float32),
            pltpu.SemaphoreType.DMA,
            pltpu.SemaphoreType.DMA,
        ],
    )
    def gather_sum(ts_hbm, td_hbm, si_hbm, di_hbm, g_hbm,
                   si_v, di_v, rs_v, rd_v, sem_s, sem_d):
        wid = lax.axis_index("c") * NS + lax.axis_index("s")

        @pl.loop(0, steps)
        def _(t):
            r = wid + t * NW

            @pl.when(r < R)
            def _():
                pltpu.sync_copy(si_hbm.at[r], si_v)
                pltpu.sync_copy(di_hbm.at[r], di_v)
                cp_s = pltpu.async_copy(ts_hbm.at[si_v], rs_v, sem_s)
                cp_d = pltpu.async_copy(td_hbm.at[di_v], rd_v, sem_d)
                cp_s.wait()
                cp_d.wait()

                @pl.loop(0, GRP)
                def _(i):
                    for j in range(H // 16):
                        sl = (i, pl.ds(j * 16, 16))
                        rs_v[sl] = rs_v[sl] + rd_v[sl]

                pltpu.sync_copy(rs_v, g_hbm.at[r])

    return gather_sum


def kernel(efeat, src_feat, dst_feat, src_idx, dst_idx, W_e, W_s, W_d, b,
           W_out, b_out, gamma, beta):
    E, EF = efeat.shape
    N, D = src_feat.shape
    H = W_s.shape[0]
    OUT = W_out.shape[0]

    NB = 2000
    tables = pl.pallas_call(
        _tables_body,
        grid=(N // NB,),
        in_specs=[
            pl.BlockSpec((NB, D), lambda i: (i, 0)),
            pl.BlockSpec((NB, D), lambda i: (i, 0)),
            pl.BlockSpec((D, H), lambda i: (0, 0)),
            pl.BlockSpec((D, H), lambda i: (0, 0)),
            pl.BlockSpec((1, H), lambda i: (0, 0)),
        ],
        out_specs=[
            pl.BlockSpec((NB, H), lambda i: (i, 0)),
            pl.BlockSpec((NB, H), lambda i: (i, 0)),
        ],
        out_shape=[
            jax.ShapeDtypeStruct((N, H), jnp.float32),
            jax.ShapeDtypeStruct((N, H), jnp.float32),
        ],
    )
    T_s, T_d = tables(src_feat, dst_feat, W_s.T, W_d.T, b.reshape(1, H))

    R = E // GRP
    si = src_idx.astype(jnp.int32).reshape(R, GRP)
    di = dst_idx.astype(jnp.int32).reshape(R, GRP)
    g = _sc_gather_sum(E, H)(T_s, T_d, si, di)
    g = g.reshape(E, H)

    BE = 3200
    out = pl.pallas_call(
        _edge_body,
        grid=(E // BE,),
        in_specs=[
            pl.BlockSpec((BE, EF), lambda i: (i, 0)),
            pl.BlockSpec((BE, H), lambda i: (i, 0)),
            pl.BlockSpec((EF, H), lambda i: (0, 0)),
            pl.BlockSpec((H, OUT), lambda i: (0, 0)),
            pl.BlockSpec((1, OUT), lambda i: (0, 0)),
            pl.BlockSpec((1, OUT), lambda i: (0, 0)),
            pl.BlockSpec((1, OUT), lambda i: (0, 0)),
        ],
        out_specs=pl.BlockSpec((BE, OUT), lambda i: (i, 0)),
        out_shape=jax.ShapeDtypeStruct((E, OUT), jnp.float32),
    )(efeat, g, W_e.T, W_out.T, b_out.reshape(1, OUT),
      gamma.reshape(1, OUT), beta.reshape(1, OUT))
    return out
